# baseline (speedup 1.0000x reference)
import functools

import jax
import jax.numpy as jnp
from jax import lax
from jax.experimental import pallas as pl
from jax.experimental.pallas import tpu as pltpu
from jax.experimental.pallas import tpu_sc as plsc

B, INC, INN, OUTC, OUTN, D = 4, 128, 100000, 128, 25000, 16
NP = B * INC
NC_SC, NS_SC, L = 2, 16, 16
NW = NC_SC * NS_SC
PPW = NP // NW
OUTN_PAD = 25168
NCH = 208
WCH = 2000
MUL_UNROLL = 5


def _sc_body(x_hbm, w_hbm, adjt_hbm, red_hbm, hplane, wbuf, adjbuf, outbuf, adj_sh):
    cid = lax.axis_index("c")
    sid = lax.axis_index("s")
    wid = sid * NC_SC + cid

    @pl.when(sid == 0)
    def _():
        pltpu.sync_copy(adjt_hbm, adj_sh)

    plsc.subcore_barrier()

    def plane_body(pi, _):
        p = wid * PPW + pi
        i = lax.rem(p, INC)
        pltpu.sync_copy(x_hbm.at[pl.ds(p * INN, INN)], hplane)

        def wchunk_body(c, _):
            pltpu.sync_copy(w_hbm.at[pl.ds(i * INN + c * WCH, WCH)], wbuf)

            def mul_body(k, _):
                base = c * WCH + k * (16 * MUL_UNROLL)
                wb = k * (16 * MUL_UNROLL)
                for u in range(MUL_UNROLL):
                    hplane[pl.ds(base + u * 16, 16)] = (
                        hplane[pl.ds(base + u * 16, 16)]
                        * wbuf[pl.ds(wb + u * 16, 16)]
                    )
                return 0

            lax.fori_loop(0, WCH // (16 * MUL_UNROLL), mul_body, 0)
            return 0

        lax.fori_loop(0, INN // WCH, wchunk_body, 0)

        def chunk_body(c, _):
            pltpu.sync_copy(adj_sh.at[pl.ds(c * (D * NCH), D * NCH)], adjbuf)

            def grp(g, _):
                nb = g * 16
                idx = adjbuf[pl.ds(nb, 16)]
                acc = plsc.load_gather(hplane, [idx])
                for d in range(1, D):
                    idx = adjbuf[pl.ds(d * NCH + nb, 16)]
                    acc = jnp.maximum(acc, plsc.load_gather(hplane, [idx]))
                outbuf[pl.ds(nb, 16)] = acc
                return 0

            lax.fori_loop(0, NCH // 16, grp, 0)
            pltpu.sync_copy(outbuf, red_hbm.at[pl.ds(p * OUTN_PAD + c * NCH, NCH)])
            return 0

        lax.fori_loop(0, OUTN_PAD // NCH, chunk_body, 0)
        return 0

    lax.fori_loop(0, PPW, plane_body, 0)


_sc_call = functools.partial(
    pl.kernel,
    out_type=jax.ShapeDtypeStruct((NP * OUTN_PAD,), jnp.float32),
    mesh=plsc.VectorSubcoreMesh(core_axis_name="c", subcore_axis_name="s"),
    scratch_types=[
        pltpu.VMEM((INN,), jnp.float32),
        pltpu.VMEM((WCH,), jnp.float32),
        pltpu.VMEM((D * NCH,), jnp.int32),
        pltpu.VMEM((NCH,), jnp.float32),
        pltpu.VMEM_SHARED((D * OUTN_PAD,), jnp.int32),
    ],
    compiler_params=pltpu.CompilerParams(needs_layout_passes=False),
)(_sc_body)


NBT = 512


def _tc_body(red_ref, ft_ref, bias_ref, out_ref):
    red = red_ref[0]
    ft = ft_ref[...]
    r = lax.dot_general(ft, red, (((0,), (0,)), ((), ())),
                        preferred_element_type=jnp.float32)
    out_ref[0] = r + bias_ref[0]


def _tc_call(red3, ft, bias):
    grid = (B, (OUTN + NBT - 1) // NBT)
    return pl.pallas_call(
        _tc_body,
        grid=grid,
        in_specs=[
            pl.BlockSpec((1, INC, NBT), lambda b, n: (b, 0, n)),
            pl.BlockSpec((INC, OUTC), lambda b, n: (0, 0)),
            pl.BlockSpec((1, OUTC, NBT), lambda b, n: (0, 0, n)),
        ],
        out_specs=pl.BlockSpec((1, OUTC, NBT), lambda b, n: (b, 0, n)),
        out_shape=jax.ShapeDtypeStruct((B, OUTC, OUTN), jnp.float32),
    )(red3, ft, bias)


def kernel(x, adj, nf_weight, ft_weight, bias):
    x2 = x.reshape(NP * INN)
    adj_pad = jnp.zeros((OUTN_PAD, D), jnp.int32).at[:OUTN].set(adj)
    adjt = adj_pad.reshape(OUTN_PAD // NCH, NCH, D).transpose(0, 2, 1).reshape(-1)
    red = _sc_call(x2, nf_weight.reshape(INC * INN), adjt)
    out = _tc_call(red.reshape(B, INC, OUTN_PAD), ft_weight, bias)
    return out


# speedup vs baseline: 1.9317x; 1.9317x over previous
import functools

import jax
import jax.numpy as jnp
from jax import lax
from jax.experimental import pallas as pl
from jax.experimental.pallas import tpu as pltpu
from jax.experimental.pallas import tpu_sc as plsc

B, INC, INN, OUTC, OUTN, D = 4, 128, 100000, 128, 25000, 16
BC = B * INC
NC_SC, NS_SC = 2, 16
NW = NC_SC * NS_SC
OUTN_PAD = 25600
NPW = OUTN_PAD // NW
G = 4
GI = G * D
NCHUNK = NPW // G
IPW = NPW * D


NB1 = 1024


def _mulT_body(x_ref, w_ref, out_ref):
    b = pl.program_id(1)
    h = x_ref[0] * w_ref[...]
    eye = jnp.eye(INC, dtype=jnp.float32)
    t = lax.dot_general(h, eye, (((0,), (0,)), ((), ())),
                        preferred_element_type=jnp.float32)
    out_ref[:, pl.ds(b * INC, INC)] = t


def _mulT(x, w):
    grid = (pl.cdiv(INN, NB1), B)
    return pl.pallas_call(
        _mulT_body,
        grid=grid,
        in_specs=[
            pl.BlockSpec((1, INC, NB1), lambda n, b: (b, 0, n)),
            pl.BlockSpec((INC, NB1), lambda n, b: (0, n)),
        ],
        out_specs=pl.BlockSpec((NB1, BC), lambda n, b: (n, 0)),
        out_shape=jax.ShapeDtypeStruct((INN, BC), jnp.float32),
    )(x, w)


def _sc_body(ht_hbm, adj_hbm, red_hbm,
             idx_all, rows0, rows1, red0, red1, sg0, sg1, so0, so1):
    cid = lax.axis_index("c")
    sid = lax.axis_index("s")
    wid = sid * NC_SC + cid
    nbase = wid * NPW

    pltpu.sync_copy(adj_hbm.at[pl.ds(wid * IPW, IPW)], idx_all)

    def gather_start(chunk, rows, sem):
        idx = idx_all.at[pl.ds(chunk * GI, GI)]
        pltpu.async_copy(ht_hbm.at[idx], rows, sem)

    def gather_wait(rows, sem):
        pltpu.make_async_copy(ht_hbm.at[idx_all.at[pl.ds(0, GI)]], rows,
                              sem).wait()

    def out_start(chunk, red, sem):
        pltpu.async_copy(red, red_hbm.at[pl.ds(nbase + chunk * G, G)], sem)

    def out_wait(red, sem):
        pltpu.make_async_copy(red, red_hbm.at[pl.ds(nbase, G)], sem).wait()

    def reduce_chunk(rows, red):
        def cbody(c, _):
            off = c * 16
            for g in range(G):
                acc = rows[g * D, pl.ds(off, 16)]
                for d in range(1, D):
                    acc = jnp.maximum(acc, rows[g * D + d, pl.ds(off, 16)])
                red[g, pl.ds(off, 16)] = acc
            return 0

        lax.fori_loop(0, BC // 16, cbody, 0)

    gather_start(0, rows0, sg0)
    gather_start(1, rows1, sg1)

    def pair(p, _):
        c0 = 2 * p

        gather_wait(rows0, sg0)

        @pl.when(p > 0)
        def _():
            out_wait(red0, so0)

        reduce_chunk(rows0, red0)
        out_start(c0, red0, so0)

        @pl.when(c0 + 2 < NCHUNK)
        def _():
            gather_start(c0 + 2, rows0, sg0)

        gather_wait(rows1, sg1)

        @pl.when(p > 0)
        def _():
            out_wait(red1, so1)

        reduce_chunk(rows1, red1)
        out_start(c0 + 1, red1, so1)

        @pl.when(c0 + 3 < NCHUNK)
        def _():
            gather_start(c0 + 3, rows1, sg1)

        return 0

    lax.fori_loop(0, NCHUNK // 2, pair, 0)
    out_wait(red0, so0)
    out_wait(red1, so1)


_sc_call = functools.partial(
    pl.kernel,
    out_type=jax.ShapeDtypeStruct((OUTN_PAD, BC), jnp.float32),
    mesh=plsc.VectorSubcoreMesh(core_axis_name="c", subcore_axis_name="s"),
    scratch_types=[
        pltpu.VMEM((IPW,), jnp.int32),
        pltpu.VMEM((GI, BC), jnp.float32),
        pltpu.VMEM((GI, BC), jnp.float32),
        pltpu.VMEM((G, BC), jnp.float32),
        pltpu.VMEM((G, BC), jnp.float32),
        pltpu.SemaphoreType.DMA,
        pltpu.SemaphoreType.DMA,
        pltpu.SemaphoreType.DMA,
        pltpu.SemaphoreType.DMA,
    ],
    compiler_params=pltpu.CompilerParams(needs_layout_passes=False),
)(_sc_body)


NB3 = 1024


def _dot_body(red_ref, ft_ref, bias_ref, out_ref):
    b = pl.program_id(1)
    red = red_ref[:, pl.ds(b * INC, INC)]
    t = lax.dot_general(ft_ref[...], red, (((0,), (1,)), ((), ())),
                        preferred_element_type=jnp.float32)
    out_ref[0] = t + bias_ref[0]


def _dot(red, ft, bias):
    grid = (pl.cdiv(OUTN, NB3), B)
    return pl.pallas_call(
        _dot_body,
        grid=grid,
        in_specs=[
            pl.BlockSpec((NB3, BC), lambda n, b: (n, 0)),
            pl.BlockSpec((INC, OUTC), lambda n, b: (0, 0)),
            pl.BlockSpec((1, OUTC, NB3), lambda n, b: (0, 0, n)),
        ],
        out_specs=pl.BlockSpec((1, OUTC, NB3), lambda n, b: (b, 0, n)),
        out_shape=jax.ShapeDtypeStruct((B, OUTC, OUTN), jnp.float32),
    )(red, ft, bias)


def kernel(x, adj, nf_weight, ft_weight, bias):
    ht = _mulT(x, nf_weight)
    adj_pad = jnp.zeros((OUTN_PAD, D), jnp.int32).at[:OUTN].set(adj)
    red = _sc_call(ht, adj_pad.reshape(-1))
    out = _dot(red, ft_weight, bias)
    return out
